# Initial kernel scaffold; baseline (speedup 1.0000x reference)
import functools

import jax
import jax.numpy as jnp
from jax import lax
from jax.experimental import pallas as pl
from jax.experimental.pallas import tpu as pltpu
from jax.experimental.pallas import tpu_sc as plsc

VOCAB = 1000000
EMBED_DIM = 32
BATCH = 16384
HIST = 50
TOTAL = BATCH * HIST

_info = plsc.get_sparse_core_info()
NUM_CORES = _info.num_cores
NUM_SUBCORES = _info.num_subcores
NW = NUM_CORES * NUM_SUBCORES
ROWS_PER_W = TOTAL // NW
CHUNK = 1024
NCHUNKS = ROWS_PER_W // CHUNK


def _gather_body(idx_hbm, table_hbm, out_hbm, idx_v, rows_v, sem):
    wid = lax.axis_index("s") * NUM_CORES + lax.axis_index("c")
    base = wid * ROWS_PER_W

    def body(i, carry):
        off = base + i * CHUNK
        pltpu.sync_copy(idx_hbm.at[pl.ds(off, CHUNK)], idx_v)
        pltpu.async_copy(table_hbm.at[idx_v], rows_v, sem).wait()
        pltpu.sync_copy(rows_v, out_hbm.at[pl.ds(off, CHUNK)])
        return carry

    lax.fori_loop(0, NCHUNKS, body, 0)


@jax.jit
def _embed_lookup(indices_flat, table):
    mesh = plsc.VectorSubcoreMesh(core_axis_name="c", subcore_axis_name="s")
    k = functools.partial(
        pl.kernel,
        mesh=mesh,
        out_type=jax.ShapeDtypeStruct((TOTAL, EMBED_DIM), jnp.float32),
        scratch_types=[
            pltpu.VMEM((CHUNK,), jnp.int32),
            pltpu.VMEM((CHUNK, EMBED_DIM), jnp.float32),
            pltpu.SemaphoreType.DMA,
        ],
    )(_gather_body)
    return k(indices_flat, table)


def kernel(indices, table):
    idx_flat = indices.reshape(TOTAL).astype(jnp.int32)
    out = _embed_lookup(idx_flat, table)
    return out.reshape(BATCH, HIST, EMBED_DIM)


# speedup vs baseline: 1.0936x; 1.0936x over previous
import functools

import jax
import jax.numpy as jnp
from jax import lax
from jax.experimental import pallas as pl
from jax.experimental.pallas import tpu as pltpu
from jax.experimental.pallas import tpu_sc as plsc

VOCAB = 1000000
EMBED_DIM = 32
BATCH = 16384
HIST = 50
TOTAL = BATCH * HIST

_info = plsc.get_sparse_core_info()
NUM_CORES = _info.num_cores
NUM_SUBCORES = _info.num_subcores
NW = NUM_CORES * NUM_SUBCORES
ROWS_PER_W = TOTAL // NW
CHUNK = 1024
NCHUNKS = ROWS_PER_W // CHUNK


def _gather_body(idx_hbm, table_hbm, out_hbm, idx_v, rows_v, sem):
    wid = lax.axis_index("s") * NUM_CORES + lax.axis_index("c")
    base = wid * ROWS_PER_W

    def body(i, carry):
        off = base + i * CHUNK
        pltpu.sync_copy(idx_hbm.at[pl.ds(off, CHUNK)], idx_v)
        pltpu.async_copy(table_hbm.at[idx_v], rows_v, sem).wait()
        pltpu.sync_copy(rows_v, out_hbm.at[pl.ds(off, CHUNK)])
        return carry

    lax.fori_loop(0, NCHUNKS, body, 0)


@jax.jit
def _embed_lookup(indices_flat, table):
    mesh = plsc.VectorSubcoreMesh(core_axis_name="c", subcore_axis_name="s")
    k = functools.partial(
        pl.kernel,
        mesh=mesh,
        out_type=jax.ShapeDtypeStruct((TOTAL, EMBED_DIM), jnp.float32),
        scratch_types=[
            pltpu.VMEM((CHUNK,), jnp.int32),
            pltpu.VMEM((CHUNK, EMBED_DIM), jnp.float32),
            pltpu.SemaphoreType.DMA,
        ],
        compiler_params=pltpu.CompilerParams(use_tc_tiling_on_sc=False),
    )(_gather_body)
    return k(indices_flat, table)


def kernel(indices, table):
    idx_flat = indices.reshape(TOTAL).astype(jnp.int32)
    out = _embed_lookup(idx_flat, table)
    return out.reshape(BATCH, HIST, EMBED_DIM)


# speedup vs baseline: 1.1124x; 1.0172x over previous
import functools

import jax
import jax.numpy as jnp
from jax import lax
from jax.experimental import pallas as pl
from jax.experimental.pallas import tpu as pltpu
from jax.experimental.pallas import tpu_sc as plsc

VOCAB = 1000000
EMBED_DIM = 32
BATCH = 16384
HIST = 50
TOTAL = BATCH * HIST

_info = plsc.get_sparse_core_info()
NUM_CORES = _info.num_cores
NUM_SUBCORES = _info.num_subcores
NW = NUM_CORES * NUM_SUBCORES
ROWS_PER_W = TOTAL // NW
CHUNK = 640
NBUF = 4
NCHUNKS = ROWS_PER_W // CHUNK
NGROUPS = NCHUNKS // NBUF


def _gather_body(idx_hbm, table_hbm, out_hbm, idx_v, rows_v, gsem):
    wid = lax.axis_index("s") * NUM_CORES + lax.axis_index("c")
    base = wid * ROWS_PER_W
    pltpu.sync_copy(idx_hbm.at[pl.ds(base, ROWS_PER_W)], idx_v)

    def start_gather(i, b):
        pltpu.async_copy(
            table_hbm.at[idx_v.at[pl.ds(i * CHUNK, CHUNK)]],
            rows_v.at[b],
            gsem.at[b],
        )

    def wait_gather(i, b):
        pltpu.make_async_copy(
            table_hbm.at[idx_v.at[pl.ds(i * CHUNK, CHUNK)]],
            rows_v.at[b],
            gsem.at[b],
        ).wait()

    def writeback(i, b):
        pltpu.sync_copy(rows_v.at[b], out_hbm.at[pl.ds(base + i * CHUNK, CHUNK)])

    for b in range(NBUF):
        start_gather(b, b)

    def group_body(g, carry):
        for b in range(NBUF):
            i = g * NBUF + b
            wait_gather(i, b)
            writeback(i, b)
            start_gather(i + NBUF, b)
        return carry

    lax.fori_loop(0, NGROUPS - 1, group_body, 0)

    for b in range(NBUF):
        i = (NGROUPS - 1) * NBUF + b
        wait_gather(i, b)
        writeback(i, b)


@jax.jit
def _embed_lookup(indices_flat, table):
    mesh = plsc.VectorSubcoreMesh(core_axis_name="c", subcore_axis_name="s")
    k = functools.partial(
        pl.kernel,
        mesh=mesh,
        out_type=jax.ShapeDtypeStruct((TOTAL, EMBED_DIM), jnp.float32),
        scratch_types=[
            pltpu.VMEM((ROWS_PER_W,), jnp.int32),
            pltpu.VMEM((NBUF, CHUNK, EMBED_DIM), jnp.float32),
            pltpu.SemaphoreType.DMA((NBUF,)),
        ],
        compiler_params=pltpu.CompilerParams(use_tc_tiling_on_sc=False),
    )(_gather_body)
    return k(indices_flat, table)


def kernel(indices, table):
    idx_flat = indices.reshape(TOTAL).astype(jnp.int32)
    out = _embed_lookup(idx_flat, table)
    return out.reshape(BATCH, HIST, EMBED_DIM)
